# baseline (speedup 1.0000x reference)
import functools
import itertools

import jax
import jax.numpy as jnp
from jax import lax
from jax.experimental import pallas as pl
from jax.experimental.pallas import tpu as pltpu
from jax.experimental.pallas import tpu_sc as plsc

DIM = 3
KS = 4
K = KS ** DIM
KT = K + 1
F = 128
N = 10000
NC = 2
NS = 16
NW = NC * NS
CH = 16
S = 8
NDEG = 10240
NACC = 10240
BN = 400


def _xw_body(x_ref, w_ref, o_ref):
    o_ref[0] = jnp.dot(x_ref[...], w_ref[0],
                       preferred_element_type=jnp.float32)


def _compute_xw(x, weight):
    return pl.pallas_call(
        _xw_body,
        grid=(N // BN, KT),
        in_specs=[
            pl.BlockSpec((BN, F), lambda nb, k: (nb, 0)),
            pl.BlockSpec((1, F, F), lambda nb, k: (k, 0, 0)),
        ],
        out_specs=pl.BlockSpec((1, BN, F), lambda nb, k: (k, nb, 0)),
        out_shape=jax.ShapeDtypeStruct((KT, N, F), jnp.float32),
    )(x, weight)


def _splat(vec, lane):
    idx = jnp.full((16, 1), lane, jnp.int32)
    dnums = lax.GatherDimensionNumbers(
        offset_dims=(), collapsed_slice_dims=(0,), start_index_map=(0,))
    return lax.gather(vec, idx, dnums, (1,),
                      mode=lax.GatherScatterMode.PROMISE_IN_BOUNDS)


def _make_sc_kernel(E, ept_pad, n_chunks):
    mesh = plsc.VectorSubcoreMesh(core_axis_name="c", subcore_axis_name="s")
    rows_per_sub = NACC // NS
    zrows = 128
    deg_per_sub = NDEG // NS

    @functools.partial(
        pl.kernel,
        out_type=(
            jax.ShapeDtypeStruct((NC, NACC, F), jnp.float32),
            jax.ShapeDtypeStruct((NC, NDEG), jnp.float32),
        ),
        mesh=mesh,
        scratch_types=[
            pltpu.VMEM_SHARED((NACC, F), jnp.float32),
            pltpu.VMEM_SHARED((NDEG,), jnp.float32),
            pltpu.VMEM((CH,), jnp.float32),
            pltpu.VMEM((CH,), jnp.float32),
            pltpu.VMEM((CH,), jnp.float32),
            pltpu.VMEM((CH,), jnp.int32),
            pltpu.VMEM((CH,), jnp.int32),
            pltpu.VMEM((S * CH,), jnp.int32),
            pltpu.VMEM((S, CH), jnp.float32),
            pltpu.VMEM((CH,), jnp.float32),
            pltpu.VMEM((CH * S, F), jnp.float32),
            pltpu.VMEM((CH, F), jnp.float32),
            pltpu.VMEM((zrows, F), jnp.float32),
            pltpu.VMEM((deg_per_sub,), jnp.float32),
            pltpu.SemaphoreType.DMA,
        ],
    )
    def sc_kernel(p0_hbm, p1_hbm, p2_hbm, row_hbm, col_hbm, xw_hbm,
                  acc_out, deg_out,
                  accum_sh, deg_sh, p0v, p1v, p2v, rowv, colv, idxv,
                  amtv, degv, rows_v, eout, zbuf, dz, gsem):
        cid = lax.axis_index("c")
        sid = lax.axis_index("s")
        wid = sid * NC + cid

        def _zloop(i, _):
            for j in range(F // 16):
                zbuf[i, pl.ds(16 * j, 16)] = jnp.zeros((16,), jnp.float32)
            return 0
        lax.fori_loop(0, zrows, _zloop, 0)

        def _dzloop(i, _):
            dz[pl.ds(i * 16, 16)] = jnp.zeros((16,), jnp.float32)
            return 0
        lax.fori_loop(0, deg_per_sub // 16, _dzloop, 0)

        for b in range(rows_per_sub // zrows):
            pltpu.sync_copy(
                zbuf, accum_sh.at[pl.ds(sid * rows_per_sub + b * zrows, zrows)])
        pltpu.sync_copy(dz, deg_sh.at[pl.ds(sid * deg_per_sub, deg_per_sub)])
        plsc.subcore_barrier()

        base0 = wid * ept_pad
        lanes = lax.iota(jnp.int32, 16)

        def _chunk(ch, _):
            base = base0 + ch * CH
            pltpu.sync_copy(p0_hbm.at[pl.ds(base, CH)], p0v)
            pltpu.sync_copy(p1_hbm.at[pl.ds(base, CH)], p1v)
            pltpu.sync_copy(p2_hbm.at[pl.ds(base, CH)], p2v)
            pltpu.sync_copy(row_hbm.at[pl.ds(base, CH)], rowv)
            pltpu.sync_copy(col_hbm.at[pl.ds(base, CH)], colv)

            colx = colv[...]
            valid = (base + lanes) < E
            degv[...] = jnp.where(valid, 1.0, 0.0).astype(jnp.float32)

            lo, fr = [], []
            pvs = (p0v, p1v, p2v)
            for d in range(DIM):
                v = pvs[d][...] * float(KS - 1)
                li = jnp.minimum(v.astype(jnp.int32), KS - 2)
                lo.append(li)
                fr.append(v - li.astype(jnp.float32))
            base_flat = colx

            for sidx, bits in enumerate(itertools.product((0, 1), repeat=DIM)):
                amt = jnp.ones((16,), jnp.float32)
                idxl = jnp.zeros((16,), jnp.int32)
                for d, b in enumerate(bits):
                    amt = amt * (fr[d] if b else (1.0 - fr[d]))
                    idxl = idxl + (lo[d] + b) * (KS ** (DIM - 1 - d))
                amt = jnp.where(valid, amt, 0.0)
                idxv[pl.ds(sidx * CH, CH)] = base_flat + idxl * N
                amtv[sidx, :] = amt

            pltpu.async_copy(xw_hbm.at[idxv], rows_v, gsem).wait()

            av = [amtv[si, :] for si in range(S)]
            for e in range(CH):
                sp = [_splat(av[si], e) for si in range(S)]
                for f8 in range(F // 16):
                    acc = sp[0] * rows_v[0 * CH + e, pl.ds(f8 * 16, 16)]
                    for si in range(1, S):
                        acc = acc + sp[si] * rows_v[si * CH + e,
                                                    pl.ds(f8 * 16, 16)]
                    eout[e, pl.ds(f8 * 16, 16)] = acc

            pltpu.sync_copy(eout, accum_sh.at[rowv], add=True)
            pltpu.sync_copy(degv, deg_sh.at[rowv], add=True)
            return 0

        lax.fori_loop(0, n_chunks, _chunk, 0)
        plsc.subcore_barrier()

        for b in range(rows_per_sub // zrows):
            off = sid * rows_per_sub + b * zrows
            pltpu.sync_copy(accum_sh.at[pl.ds(off, zrows)],
                            acc_out.at[cid, pl.ds(off, zrows)])
        pltpu.sync_copy(deg_sh.at[pl.ds(sid * deg_per_sub, deg_per_sub)],
                        deg_out.at[cid, pl.ds(sid * deg_per_sub, deg_per_sub)])

    return sc_kernel


def _combine_body(a_ref, d_ref, r_ref, b_ref, o_ref):
    a = a_ref[0] + a_ref[1]
    d = d_ref[0] + d_ref[1]
    o_ref[...] = a / jnp.maximum(d, 1.0) + r_ref[...] + b_ref[...]


def _combine(acc, deg, root, bias):
    return pl.pallas_call(
        _combine_body,
        grid=(N // BN,),
        in_specs=[
            pl.BlockSpec((NC, BN, F), lambda i: (0, i, 0)),
            pl.BlockSpec((NC, BN, 1), lambda i: (0, i, 0)),
            pl.BlockSpec((BN, F), lambda i: (i, 0)),
            pl.BlockSpec((1, F), lambda i: (0, 0)),
        ],
        out_specs=pl.BlockSpec((BN, F), lambda i: (i, 0)),
        out_shape=jax.ShapeDtypeStruct((N, F), jnp.float32),
    )(acc, deg, root, bias)


def kernel(x, edge_index, pseudo, weight, bias):
    E = edge_index.shape[1]
    ept = -(-E // NW)
    n_chunks = -(-ept // CH)
    ept_pad = n_chunks * CH
    e_pad = ept_pad * NW

    xw = _compute_xw(x, weight)
    xw_flat = xw.reshape(KT * N, F)
    root = xw[K]

    row = edge_index[0]
    col = edge_index[1]
    pad = e_pad - E
    rowp = jnp.pad(row, (0, pad))
    colp = jnp.pad(col, (0, pad))
    pp = [jnp.pad(pseudo[:, d], (0, pad)) for d in range(DIM)]

    sc = _make_sc_kernel(E, ept_pad, n_chunks)
    acc, deg = sc(pp[0], pp[1], pp[2], rowp, colp, xw_flat)

    deg3 = deg[:, :N].reshape(NC, N, 1)
    return _combine(acc, deg3, root, bias.reshape(1, F))


# speedup vs baseline: 1.7092x; 1.7092x over previous
import functools
import itertools

import jax
import jax.numpy as jnp
from jax import lax
from jax.experimental import pallas as pl
from jax.experimental.pallas import tpu as pltpu
from jax.experimental.pallas import tpu_sc as plsc

DIM = 3
KS = 4
K = KS ** DIM
KT = K + 1
F = 128
N = 10000
NC = 2
NS = 16
NW = NC * NS
CH = 16
S = 8
GPS = 8
SCB = GPS * CH
NDEG = 10240
NACC = 10240
BN = 400


def _xw_body(x_ref, w_ref, o_ref):
    o_ref[0] = jnp.dot(x_ref[...].astype(jnp.bfloat16),
                       w_ref[0].astype(jnp.bfloat16),
                       preferred_element_type=jnp.float32)


def _compute_xw(x, weight):
    return pl.pallas_call(
        _xw_body,
        grid=(N // BN, KT),
        in_specs=[
            pl.BlockSpec((BN, F), lambda nb, k: (nb, 0)),
            pl.BlockSpec((1, F, F), lambda nb, k: (k, 0, 0)),
        ],
        out_specs=pl.BlockSpec((1, BN, F), lambda nb, k: (k, nb, 0)),
        out_shape=jax.ShapeDtypeStruct((KT, N, F), jnp.float32),
    )(x, weight)


def _splat(vec, lane):
    idx = jnp.full((16, 1), lane, jnp.int32)
    dnums = lax.GatherDimensionNumbers(
        offset_dims=(), collapsed_slice_dims=(0,), start_index_map=(0,))
    return lax.gather(vec, idx, dnums, (1,),
                      mode=lax.GatherScatterMode.PROMISE_IN_BOUNDS)


def _make_sc_kernel(E, ept_pad, nsc):
    mesh = plsc.VectorSubcoreMesh(core_axis_name="c", subcore_axis_name="s")
    rows_per_sub = NACC // NS
    zrows = 128
    deg_per_sub = NDEG // NS

    @functools.partial(
        pl.kernel,
        out_type=(
            jax.ShapeDtypeStruct((NC, NACC, F), jnp.float32),
            jax.ShapeDtypeStruct((NC, NDEG), jnp.float32),
        ),
        mesh=mesh,
        scratch_types=[
            pltpu.VMEM_SHARED((NACC, F), jnp.float32),
            pltpu.VMEM_SHARED((NDEG,), jnp.float32),
            pltpu.VMEM((2 * SCB,), jnp.float32),
            pltpu.VMEM((2 * SCB,), jnp.float32),
            pltpu.VMEM((2 * SCB,), jnp.float32),
            pltpu.VMEM((2, GPS, CH), jnp.int32),
            pltpu.VMEM((2 * SCB,), jnp.int32),
            pltpu.VMEM((2 * SCB * S,), jnp.int32),
            pltpu.VMEM((2, S, SCB), jnp.float32),
            pltpu.VMEM((2 * SCB,), jnp.float32),
            pltpu.VMEM((SCB // 128, 128), jnp.int32),
            pltpu.VMEM((2, CH * S, F), jnp.float32),
            pltpu.VMEM((2, CH, F), jnp.float32),
            pltpu.VMEM((deg_per_sub,), jnp.float32),
            pltpu.SemaphoreType.DMA,
            pltpu.SemaphoreType.DMA,
            pltpu.SemaphoreType.DMA,
            pltpu.SemaphoreType.DMA,
            pltpu.SemaphoreType.DMA,
            pltpu.SemaphoreType.DMA,
        ],
    )
    def sc_kernel(p0_hbm, p1_hbm, p2_hbm, row_hbm, col_hbm, xw_hbm,
                  acc_out, deg_out,
                  accum_sh, deg_sh, p0v, p1v, p2v, rowv, colv, idxv,
                  amtv, degv, rowscat128, rows2, eout2, dz,
                  isem, gsem0, gsem1, ssem0, ssem1, dsem):
        cid = lax.axis_index("c")
        sid = lax.axis_index("s")
        wid = sid * NC + cid
        gsems = (gsem0, gsem1)
        ssems = (ssem0, ssem1)

        def _zloop(i, _):
            for j in range(F // 16):
                rows2[0, i, pl.ds(16 * j, 16)] = jnp.zeros((16,), jnp.float32)
            return 0
        lax.fori_loop(0, zrows, _zloop, 0)

        def _dzloop(i, _):
            dz[pl.ds(i * 16, 16)] = jnp.zeros((16,), jnp.float32)
            return 0
        lax.fori_loop(0, deg_per_sub // 16, _dzloop, 0)

        for b in range(rows_per_sub // zrows):
            pltpu.sync_copy(
                rows2.at[0],
                accum_sh.at[pl.ds(sid * rows_per_sub + b * zrows, zrows)])
        pltpu.sync_copy(dz, deg_sh.at[pl.ds(sid * deg_per_sub, deg_per_sub)])
        plsc.subcore_barrier()

        base0 = wid * ept_pad
        rbase0 = (wid * ept_pad) // CH
        lanes = lax.iota(jnp.int32, 16)
        inps = ((p0_hbm, p0v), (p1_hbm, p1v), (p2_hbm, p2v),
                (col_hbm, colv))

        def _issue_inputs(i):
            off = base0 + i * SCB
            nb = i % 2
            for hbm, buf in inps:
                pltpu.async_copy(hbm.at[pl.ds(off, SCB)],
                                 buf.at[pl.ds(nb * SCB, SCB)], isem)
            pltpu.async_copy(
                row_hbm.at[pl.ds(pl.multiple_of(rbase0 + i * GPS, 8), GPS), :],
                rowv.at[nb], isem)

        def _wait_inputs(i):
            off = base0 + i * SCB
            nb = i % 2
            for hbm, buf in inps:
                pltpu.make_async_copy(
                    hbm.at[pl.ds(off, SCB)],
                    buf.at[pl.ds(nb * SCB, SCB)], isem).wait()
            pltpu.make_async_copy(
                row_hbm.at[pl.ds(pl.multiple_of(rbase0 + i * GPS, 8), GPS), :],
                rowv.at[nb], isem).wait()

        def _basis(i, nb, g):
            e0 = g * CH
            colx = colv[pl.ds(nb * SCB + e0, CH)]
            valid = (base0 + i * SCB + e0 + lanes) < E
            degv[pl.ds(nb * SCB + e0, CH)] = jnp.where(valid, 1.0, 0.0).astype(
                jnp.float32)
            lo, fr = [], []
            for pv in (p0v, p1v, p2v):
                v = pv[pl.ds(nb * SCB + e0, CH)] * float(KS - 1)
                li = jnp.minimum(v.astype(jnp.int32), KS - 2)
                lo.append(li)
                fr.append(v - li.astype(jnp.float32))
            for sidx, bits in enumerate(itertools.product((0, 1), repeat=DIM)):
                amt = jnp.ones((16,), jnp.float32)
                idxl = jnp.zeros((16,), jnp.int32)
                for d, bit in enumerate(bits):
                    amt = amt * (fr[d] if bit else (1.0 - fr[d]))
                    idxl = idxl + (lo[d] + bit) * (KS ** (DIM - 1 - d))
                amt = jnp.where(valid, amt, 0.0)
                idxv[pl.ds(nb * (SCB * S) + g * (CH * S) + sidx * CH, CH)] = idxl * N + colx
                amtv[nb, sidx, pl.ds(e0, CH)] = amt

        def _gather_desc(nb, g, p):
            return pltpu.make_async_copy(
                xw_hbm.at[idxv.at[pl.ds(nb * (SCB * S) + g * (CH * S), CH * S)]],
                rows2.at[p], gsems[p])

        def _scat_desc(b, g, p):
            return pltpu.make_async_copy(
                eout2.at[p], accum_sh.at[rowv.at[b, g]], ssems[p])

        def _compute(b, g, p):
            av = [amtv[b, si, pl.ds(g * CH, CH)] for si in range(S)]

            def _edge(e, _):
                sp = [_splat(av[si], e) for si in range(S)]
                for f8 in range(F // 16):
                    acc = sp[0] * rows2[p, 0 * CH + e, pl.ds(f8 * 16, 16)]
                    for si in range(1, S):
                        acc = acc + sp[si] * rows2[p, si * CH + e,
                                                   pl.ds(f8 * 16, 16)]
                    eout2[p, e, pl.ds(f8 * 16, 16)] = acc
                return 0
            lax.fori_loop(0, CH, _edge, 0)

        def _group(i, b, gp, g, p):
            if p == 0:
                _gather_desc(b, g + 1, 1).start()
            else:
                @pl.when(gp < GPS // 2 - 1)
                def _():
                    _gather_desc(b, g + 1, 0).start()

                @pl.when(jnp.logical_and(gp == GPS // 2 - 1, i < nsc - 1))
                def _():
                    _gather_desc(1 - b, 0, 0).start()
            @pl.when(gp >= 1)
            def _():
                _scat_desc(b, g - 2, p).wait()
            _gather_desc(b, g, p).wait()
            _compute(b, g, p)
            pltpu.async_copy(eout2.at[p], accum_sh.at[rowv.at[b, g]],
                             ssems[p], add=True)
            _basis(i + 1, 1 - b, g)

        _issue_inputs(0)
        _wait_inputs(0)

        def _basis0(g, _):
            _basis(0, 0, g)
            return 0
        lax.fori_loop(0, GPS, _basis0, 0)
        if nsc > 1:
            _issue_inputs(1)
        _gather_desc(0, 0, 0).start()

        def _superchunk(i, _):
            b = i % 2
            @pl.when(i + 1 < nsc)
            def _():
                _wait_inputs(i + 1)

            @pl.when(i > 0)
            def _():
                for j in range(SCB // 128):
                    pltpu.make_async_copy(
                        degv.at[pl.ds((1 - b) * SCB + j * 128, 128)],
                        deg_sh.at[rowscat128.at[j]], dsem).wait()

            def _stage128(j, _2):
                for jj in range(8):
                    rowscat128[j, pl.ds(jj * 16, 16)] = rowv[b, j * 8 + jj, :]
                return 0
            lax.fori_loop(0, SCB // 128, _stage128, 0)

            for j in range(SCB // 128):
                pltpu.async_copy(degv.at[pl.ds(b * SCB + j * 128, 128)],
                                 deg_sh.at[rowscat128.at[j]], dsem, add=True)

            def _pair(gp, _2):
                _group(i, b, gp, 2 * gp, 0)
                _group(i, b, gp, 2 * gp + 1, 1)
                return 0
            lax.fori_loop(0, GPS // 2, _pair, 0)

            _scat_desc(b, GPS - 2, 0).wait()
            _scat_desc(b, GPS - 1, 1).wait()

            @pl.when(i + 2 < nsc)
            def _():
                _issue_inputs(i + 2)
            return 0

        lax.fori_loop(0, nsc, _superchunk, 0)
        blast = (nsc - 1) % 2
        for j in range(SCB // 128):
            pltpu.make_async_copy(
                degv.at[pl.ds(blast * SCB + j * 128, 128)],
                deg_sh.at[rowscat128.at[j]], dsem).wait()
        plsc.subcore_barrier()

        for b in range(rows_per_sub // zrows):
            off = sid * rows_per_sub + b * zrows
            pltpu.sync_copy(accum_sh.at[pl.ds(off, zrows)],
                            acc_out.at[cid, pl.ds(off, zrows)])
        pltpu.sync_copy(deg_sh.at[pl.ds(sid * deg_per_sub, deg_per_sub)],
                        deg_out.at[cid, pl.ds(sid * deg_per_sub, deg_per_sub)])

    return sc_kernel


def _combine_body(a_ref, d_ref, r_ref, b_ref, o_ref):
    a = a_ref[0] + a_ref[1]
    d = d_ref[0] + d_ref[1]
    o_ref[...] = a / jnp.maximum(d, 1.0) + r_ref[...] + b_ref[...]


def _combine(acc, deg, root, bias):
    return pl.pallas_call(
        _combine_body,
        grid=(N // BN,),
        in_specs=[
            pl.BlockSpec((NC, BN, F), lambda i: (0, i, 0)),
            pl.BlockSpec((NC, BN, 1), lambda i: (0, i, 0)),
            pl.BlockSpec((BN, F), lambda i: (i, 0)),
            pl.BlockSpec((1, F), lambda i: (0, 0)),
        ],
        out_specs=pl.BlockSpec((BN, F), lambda i: (i, 0)),
        out_shape=jax.ShapeDtypeStruct((N, F), jnp.float32),
    )(acc, deg, root, bias)


def kernel(x, edge_index, pseudo, weight, bias):
    E = edge_index.shape[1]
    ept = -(-E // NW)
    nsc = -(-ept // SCB)
    ept_pad = nsc * SCB
    e_pad = ept_pad * NW

    xw = _compute_xw(x, weight)
    xw_flat = xw.reshape(KT * N, F)
    root = xw[K]

    row = edge_index[0]
    col = edge_index[1]
    pad = e_pad - E
    rowp = jnp.pad(row, (0, pad)).reshape(e_pad // CH, CH)
    colp = jnp.pad(col, (0, pad))
    pp = [jnp.pad(pseudo[:, d], (0, pad)) for d in range(DIM)]

    sc = _make_sc_kernel(E, ept_pad, nsc)
    acc, deg = sc(pp[0], pp[1], pp[2], rowp, colp, xw_flat)

    deg3 = deg[:, :N].reshape(NC, N, 1)
    return _combine(acc, deg3, root, bias.reshape(1, F))


# speedup vs baseline: 3.2617x; 1.9083x over previous
import functools
import itertools

import jax
import jax.numpy as jnp
from jax import lax
from jax.experimental import pallas as pl
from jax.experimental.pallas import tpu as pltpu
from jax.experimental.pallas import tpu_sc as plsc

DIM = 3
KS = 4
K = KS ** DIM
KT = K + 1
F = 128
N = 10000
NC = 2
NS = 16
NW = NC * NS
CH = 16
S = 8
GPS = 8
SCB = GPS * CH
NDEG = 10240
NACC = 10240
BN = 400


def _xw_body(x_ref, w_ref, o_ref):
    xb = x_ref[...]
    for k in range(KT):
        o_ref[k] = jnp.dot(xb, w_ref[k], preferred_element_type=jnp.float32)


def _compute_xw(x, weight):
    return pl.pallas_call(
        _xw_body,
        grid=(N // BN,),
        in_specs=[
            pl.BlockSpec((BN, F), lambda nb: (nb, 0)),
            pl.BlockSpec((KT, F, F), lambda nb: (0, 0, 0)),
        ],
        out_specs=pl.BlockSpec((KT, BN, F), lambda nb: (0, nb, 0)),
        out_shape=jax.ShapeDtypeStruct((KT, N, F), jnp.float32),
    )(x.astype(jnp.bfloat16), weight.astype(jnp.bfloat16))


def _splat(vec, lane):
    idx = jnp.full((16, 1), lane, jnp.int32)
    dnums = lax.GatherDimensionNumbers(
        offset_dims=(), collapsed_slice_dims=(0,), start_index_map=(0,))
    return lax.gather(vec, idx, dnums, (1,),
                      mode=lax.GatherScatterMode.PROMISE_IN_BOUNDS)


def _make_sc_kernel(E, ept_pad, nsc):
    mesh = plsc.VectorSubcoreMesh(core_axis_name="c", subcore_axis_name="s")
    rows_per_sub = NACC // NS
    zrows = 128
    deg_per_sub = NDEG // NS

    @functools.partial(
        pl.kernel,
        out_type=(
            jax.ShapeDtypeStruct((NC, NACC, F), jnp.float32),
            jax.ShapeDtypeStruct((NC, NDEG), jnp.float32),
        ),
        mesh=mesh,
        scratch_types=[
            pltpu.VMEM_SHARED((NACC, F), jnp.float32),
            pltpu.VMEM_SHARED((NDEG,), jnp.float32),
            pltpu.VMEM((2 * SCB,), jnp.float32),
            pltpu.VMEM((2 * SCB,), jnp.float32),
            pltpu.VMEM((2 * SCB,), jnp.float32),
            pltpu.VMEM((2, GPS, CH), jnp.int32),
            pltpu.VMEM((2 * SCB,), jnp.int32),
            pltpu.VMEM((2 * SCB * S,), jnp.int32),
            pltpu.VMEM((2, S, SCB), jnp.float32),
            pltpu.VMEM((2 * SCB,), jnp.float32),
            pltpu.VMEM((SCB // 128, 128), jnp.int32),
            pltpu.VMEM((2, CH * S, F), jnp.float32),
            pltpu.VMEM((2, CH, F), jnp.float32),
            pltpu.VMEM((deg_per_sub,), jnp.float32),
            pltpu.SemaphoreType.DMA,
            pltpu.SemaphoreType.DMA,
            pltpu.SemaphoreType.DMA,
            pltpu.SemaphoreType.DMA,
            pltpu.SemaphoreType.DMA,
            pltpu.SemaphoreType.DMA,
        ],
    )
    def sc_kernel(p0_hbm, p1_hbm, p2_hbm, row_hbm, col_hbm, xw_hbm,
                  acc_out, deg_out,
                  accum_sh, deg_sh, p0v, p1v, p2v, rowv, colv, idxv,
                  amtv, degv, rowscat128, rows2, eout2, dz,
                  isem, gsem0, gsem1, ssem0, ssem1, dsem):
        cid = lax.axis_index("c")
        sid = lax.axis_index("s")
        wid = sid * NC + cid
        gsems = (gsem0, gsem1)
        ssems = (ssem0, ssem1)

        def _zloop(i, _):
            for j in range(F // 16):
                rows2[0, i, pl.ds(16 * j, 16)] = jnp.zeros((16,), jnp.float32)
            return 0
        lax.fori_loop(0, zrows, _zloop, 0)

        def _dzloop(i, _):
            dz[pl.ds(i * 16, 16)] = jnp.zeros((16,), jnp.float32)
            return 0
        lax.fori_loop(0, deg_per_sub // 16, _dzloop, 0)

        for b in range(rows_per_sub // zrows):
            pltpu.sync_copy(
                rows2.at[0],
                accum_sh.at[pl.ds(sid * rows_per_sub + b * zrows, zrows)])
        pltpu.sync_copy(dz, deg_sh.at[pl.ds(sid * deg_per_sub, deg_per_sub)])
        plsc.subcore_barrier()

        base0 = wid * ept_pad
        rbase0 = (wid * ept_pad) // CH
        lanes = lax.iota(jnp.int32, 16)
        inps = ((p0_hbm, p0v), (p1_hbm, p1v), (p2_hbm, p2v),
                (col_hbm, colv))

        def _issue_inputs(i):
            off = base0 + i * SCB
            nb = i % 2
            for hbm, buf in inps:
                pltpu.async_copy(hbm.at[pl.ds(off, SCB)],
                                 buf.at[pl.ds(nb * SCB, SCB)], isem)
            pltpu.async_copy(
                row_hbm.at[pl.ds(pl.multiple_of(rbase0 + i * GPS, 8), GPS), :],
                rowv.at[nb], isem)

        def _wait_inputs(i):
            off = base0 + i * SCB
            nb = i % 2
            for hbm, buf in inps:
                pltpu.make_async_copy(
                    hbm.at[pl.ds(off, SCB)],
                    buf.at[pl.ds(nb * SCB, SCB)], isem).wait()
            pltpu.make_async_copy(
                row_hbm.at[pl.ds(pl.multiple_of(rbase0 + i * GPS, 8), GPS), :],
                rowv.at[nb], isem).wait()

        def _basis(i, nb, g):
            e0 = g * CH
            colx = colv[pl.ds(nb * SCB + e0, CH)]
            valid = (base0 + i * SCB + e0 + lanes) < E
            degv[pl.ds(nb * SCB + e0, CH)] = jnp.where(valid, 1.0, 0.0).astype(
                jnp.float32)
            lo, fr = [], []
            for pv in (p0v, p1v, p2v):
                v = pv[pl.ds(nb * SCB + e0, CH)] * float(KS - 1)
                li = jnp.minimum(v.astype(jnp.int32), KS - 2)
                lo.append(li)
                fr.append(v - li.astype(jnp.float32))
            for sidx, bits in enumerate(itertools.product((0, 1), repeat=DIM)):
                amt = jnp.ones((16,), jnp.float32)
                idxl = jnp.zeros((16,), jnp.int32)
                for d, bit in enumerate(bits):
                    amt = amt * (fr[d] if bit else (1.0 - fr[d]))
                    idxl = idxl + (lo[d] + bit) * (KS ** (DIM - 1 - d))
                amt = jnp.where(valid, amt, 0.0)
                idxv[pl.ds(nb * (SCB * S) + g * (CH * S) + sidx * CH, CH)] = idxl * N + colx
                amtv[nb, sidx, pl.ds(e0, CH)] = amt

        def _gather_desc(nb, g, p):
            return pltpu.make_async_copy(
                xw_hbm.at[idxv.at[pl.ds(nb * (SCB * S) + g * (CH * S), CH * S)]],
                rows2.at[p], gsems[p])

        def _scat_desc(b, g, p):
            return pltpu.make_async_copy(
                eout2.at[p], accum_sh.at[rowv.at[b, g]], ssems[p])

        def _compute(b, g, p):
            av = [amtv[b, si, pl.ds(g * CH, CH)] for si in range(S)]

            def _edge(e, _):
                sp = [_splat(av[si], e) for si in range(S)]
                for f8 in range(F // 16):
                    acc = sp[0] * rows2[p, 0 * CH + e, pl.ds(f8 * 16, 16)]
                    for si in range(1, S):
                        acc = acc + sp[si] * rows2[p, si * CH + e,
                                                   pl.ds(f8 * 16, 16)]
                    eout2[p, e, pl.ds(f8 * 16, 16)] = acc
                return 0
            lax.fori_loop(0, CH, _edge, 0)

        def _group(i, b, gp, g, p):
            if p == 0:
                _gather_desc(b, g + 1, 1).start()
            else:
                @pl.when(gp < GPS // 2 - 1)
                def _():
                    _gather_desc(b, g + 1, 0).start()

                @pl.when(jnp.logical_and(gp == GPS // 2 - 1, i < nsc - 1))
                def _():
                    _gather_desc(1 - b, 0, 0).start()
            @pl.when(gp >= 1)
            def _():
                _scat_desc(b, g - 2, p).wait()
            _gather_desc(b, g, p).wait()
            _compute(b, g, p)
            pltpu.async_copy(eout2.at[p], accum_sh.at[rowv.at[b, g]],
                             ssems[p], add=True)
            _basis(i + 1, 1 - b, g)

        _issue_inputs(0)
        _wait_inputs(0)

        def _basis0(g, _):
            _basis(0, 0, g)
            return 0
        lax.fori_loop(0, GPS, _basis0, 0)
        if nsc > 1:
            _issue_inputs(1)
        _gather_desc(0, 0, 0).start()

        def _superchunk(i, _):
            b = i % 2
            @pl.when(i + 1 < nsc)
            def _():
                _wait_inputs(i + 1)

            @pl.when(i > 0)
            def _():
                for j in range(SCB // 128):
                    pltpu.make_async_copy(
                        degv.at[pl.ds((1 - b) * SCB + j * 128, 128)],
                        deg_sh.at[rowscat128.at[j]], dsem).wait()

            def _stage128(j, _2):
                for jj in range(8):
                    rowscat128[j, pl.ds(jj * 16, 16)] = rowv[b, j * 8 + jj, :]
                return 0
            lax.fori_loop(0, SCB // 128, _stage128, 0)

            for j in range(SCB // 128):
                pltpu.async_copy(degv.at[pl.ds(b * SCB + j * 128, 128)],
                                 deg_sh.at[rowscat128.at[j]], dsem, add=True)

            def _pair(gp, _2):
                _group(i, b, gp, 2 * gp, 0)
                _group(i, b, gp, 2 * gp + 1, 1)
                return 0
            lax.fori_loop(0, GPS // 2, _pair, 0)

            _scat_desc(b, GPS - 2, 0).wait()
            _scat_desc(b, GPS - 1, 1).wait()

            @pl.when(i + 2 < nsc)
            def _():
                _issue_inputs(i + 2)
            return 0

        lax.fori_loop(0, nsc, _superchunk, 0)
        blast = (nsc - 1) % 2
        for j in range(SCB // 128):
            pltpu.make_async_copy(
                degv.at[pl.ds(blast * SCB + j * 128, 128)],
                deg_sh.at[rowscat128.at[j]], dsem).wait()
        plsc.subcore_barrier()

        for b in range(rows_per_sub // zrows):
            off = sid * rows_per_sub + b * zrows
            pltpu.sync_copy(accum_sh.at[pl.ds(off, zrows)],
                            acc_out.at[cid, pl.ds(off, zrows)])
        pltpu.sync_copy(deg_sh.at[pl.ds(sid * deg_per_sub, deg_per_sub)],
                        deg_out.at[cid, pl.ds(sid * deg_per_sub, deg_per_sub)])

    return sc_kernel


def _combine_body(a_ref, d_ref, r_ref, b_ref, o_ref):
    a = a_ref[0] + a_ref[1]
    d = d_ref[0] + d_ref[1]
    o_ref[...] = a / jnp.maximum(d, 1.0) + r_ref[...] + b_ref[...]


def _combine(acc, deg, root, bias):
    return pl.pallas_call(
        _combine_body,
        grid=(N // BN,),
        in_specs=[
            pl.BlockSpec((NC, BN, F), lambda i: (0, i, 0)),
            pl.BlockSpec((NC, BN, 1), lambda i: (0, i, 0)),
            pl.BlockSpec((BN, F), lambda i: (i, 0)),
            pl.BlockSpec((1, F), lambda i: (0, 0)),
        ],
        out_specs=pl.BlockSpec((BN, F), lambda i: (i, 0)),
        out_shape=jax.ShapeDtypeStruct((N, F), jnp.float32),
    )(acc, deg, root, bias)


def kernel(x, edge_index, pseudo, weight, bias):
    E = edge_index.shape[1]
    ept = -(-E // NW)
    nsc = -(-ept // SCB)
    ept_pad = nsc * SCB
    e_pad = ept_pad * NW

    xw = _compute_xw(x, weight)
    xw_flat = xw.reshape(KT * N, F)
    root = xw[K]

    row = edge_index[0]
    col = edge_index[1]
    pad = e_pad - E
    rowp = jnp.pad(row, (0, pad)).reshape(e_pad // CH, CH)
    colp = jnp.pad(col, (0, pad))
    pp = [jnp.pad(pseudo[:, d], (0, pad)) for d in range(DIM)]

    sc = _make_sc_kernel(E, ept_pad, nsc)
    acc, deg = sc(pp[0], pp[1], pp[2], rowp, colp, xw_flat)

    deg3 = deg[:, :N].reshape(NC, N, 1)
    return _combine(acc, deg3, root, bias.reshape(1, F))
